# baseline (speedup 1.0000x reference)
import functools

import jax
import jax.numpy as jnp
from jax import lax
from jax.experimental import pallas as pl
from jax.experimental.pallas import tpu as pltpu
from jax.experimental.pallas import tpu_sc as plsc

_B = 4096
_S = 50
_NW = 32
_PER_W = (_B * _S) // _NW
_NG = 5
_GRP = _PER_W // _NG

_mesh = plsc.VectorSubcoreMesh(core_axis_name="c", subcore_axis_name="s")


@functools.partial(
    pl.kernel,
    mesh=_mesh,
    out_type=jax.ShapeDtypeStruct((_NW, _PER_W), jnp.int32),
    scratch_types=[
        pltpu.VMEM((_PER_W,), jnp.int32),
        pltpu.VMEM((_PER_W,), jnp.int32),
        pltpu.SemaphoreType.DMA((_NG,)),
        pltpu.SemaphoreType.DMA,
    ],
)
def _sc_gather(idx_hbm, table_hbm, out_hbm, idx_v, out_v, gsem, osem):
    wid = lax.axis_index("s") * 2 + lax.axis_index("c")
    pltpu.sync_copy(idx_hbm.at[wid], idx_v)

    for g in range(_NG):
        s = pl.ds(g * _GRP, _GRP)
        pltpu.async_copy(table_hbm.at[idx_v.at[s]], out_v.at[s], gsem.at[g])
    for g in range(_NG):
        s = pl.ds(g * _GRP, _GRP)
        pltpu.make_async_copy(table_hbm.at[idx_v.at[s]], out_v.at[s], gsem.at[g]).wait()
        pltpu.async_copy(out_v.at[s], out_hbm.at[wid].at[s], osem)
    for g in range(_NG):
        s = pl.ds(g * _GRP, _GRP)
        pltpu.make_async_copy(out_v.at[s], out_hbm.at[wid].at[s], osem).wait()


def kernel(tokens_list, vocab_table):
    idx = tokens_list.reshape(_NW, _PER_W)
    out = _sc_gather(idx, vocab_table)
    return out.reshape(_B, _S)


# speedup vs baseline: 1.0064x; 1.0064x over previous
import functools

import jax
import jax.numpy as jnp
from jax import lax
from jax.experimental import pallas as pl
from jax.experimental.pallas import tpu as pltpu
from jax.experimental.pallas import tpu_sc as plsc

_B = 4096
_S = 50
_NW = 32
_PER_W = (_B * _S) // _NW
_NG = 2
_GRP = _PER_W // _NG

_mesh = plsc.VectorSubcoreMesh(core_axis_name="c", subcore_axis_name="s")


@functools.partial(
    pl.kernel,
    mesh=_mesh,
    out_type=jax.ShapeDtypeStruct((_NW, _PER_W), jnp.int32),
    scratch_types=[
        pltpu.VMEM((_PER_W,), jnp.int32),
        pltpu.VMEM((_PER_W,), jnp.int32),
        pltpu.SemaphoreType.DMA((_NG,)),
        pltpu.SemaphoreType.DMA,
    ],
)
def _sc_gather(idx_hbm, table_hbm, out_hbm, idx_v, out_v, gsem, osem):
    wid = lax.axis_index("s") * 2 + lax.axis_index("c")
    pltpu.sync_copy(idx_hbm.at[wid], idx_v)

    for g in range(_NG):
        s = pl.ds(g * _GRP, _GRP)
        pltpu.async_copy(table_hbm.at[idx_v.at[s]], out_v.at[s], gsem.at[g])
    for g in range(_NG):
        s = pl.ds(g * _GRP, _GRP)
        pltpu.make_async_copy(table_hbm.at[idx_v.at[s]], out_v.at[s], gsem.at[g]).wait()
        pltpu.async_copy(out_v.at[s], out_hbm.at[wid].at[s], osem)
    for g in range(_NG):
        s = pl.ds(g * _GRP, _GRP)
        pltpu.make_async_copy(out_v.at[s], out_hbm.at[wid].at[s], osem).wait()


def kernel(tokens_list, vocab_table):
    idx = tokens_list.reshape(_NW, _PER_W)
    out = _sc_gather(idx, vocab_table)
    return out.reshape(_B, _S)


# speedup vs baseline: 1.0110x; 1.0046x over previous
import functools

import jax
import jax.numpy as jnp
from jax import lax
from jax.experimental import pallas as pl
from jax.experimental.pallas import tpu as pltpu
from jax.experimental.pallas import tpu_sc as plsc

_B = 4096
_S = 50
_NW = 32
_PER_W = (_B * _S) // _NW
_H = _PER_W // 2

_mesh = plsc.VectorSubcoreMesh(core_axis_name="c", subcore_axis_name="s")


@functools.partial(
    pl.kernel,
    mesh=_mesh,
    out_type=jax.ShapeDtypeStruct((_NW, _PER_W), jnp.int32),
    scratch_types=[
        pltpu.VMEM((_PER_W,), jnp.int32),
        pltpu.VMEM((_PER_W,), jnp.int32),
        pltpu.SemaphoreType.DMA((2,)),
    ],
)
def _sc_gather(idx_hbm, table_hbm, out_hbm, idx_v, out_v, gsem):
    wid = lax.axis_index("s") * 2 + lax.axis_index("c")
    lo = pl.ds(0, _H)
    hi = pl.ds(_H, _H)
    pltpu.sync_copy(idx_hbm.at[wid].at[lo], idx_v.at[lo])
    pltpu.async_copy(table_hbm.at[idx_v.at[lo]], out_v.at[lo], gsem.at[0])
    pltpu.sync_copy(idx_hbm.at[wid].at[hi], idx_v.at[hi])
    pltpu.async_copy(table_hbm.at[idx_v.at[hi]], out_v.at[hi], gsem.at[1])
    pltpu.make_async_copy(table_hbm.at[idx_v.at[lo]], out_v.at[lo], gsem.at[0]).wait()
    pltpu.make_async_copy(table_hbm.at[idx_v.at[hi]], out_v.at[hi], gsem.at[1]).wait()
    pltpu.sync_copy(out_v, out_hbm.at[wid])


def kernel(tokens_list, vocab_table):
    idx = tokens_list.reshape(_NW, _PER_W)
    out = _sc_gather(idx, vocab_table)
    return out.reshape(_B, _S)


# speedup vs baseline: 1.0283x; 1.0171x over previous
import functools

import jax
import jax.numpy as jnp
from jax import lax
from jax.experimental import pallas as pl
from jax.experimental.pallas import tpu as pltpu
from jax.experimental.pallas import tpu_sc as plsc

_B = 4096
_S = 50
_NW = 32
_PER_W = (_B * _S) // _NW
_CHUNK = 128
_NCH = _PER_W // _CHUNK

_mesh = plsc.VectorSubcoreMesh(core_axis_name="c", subcore_axis_name="s")


@functools.partial(
    pl.kernel,
    mesh=_mesh,
    out_type=jax.ShapeDtypeStruct((_NW, _PER_W), jnp.int32),
    scratch_types=[
        pltpu.VMEM((_PER_W,), jnp.int32),
        pltpu.VMEM((_PER_W,), jnp.int32),
        pltpu.SemaphoreType.DMA,
    ],
)
def _sc_gather(idx_hbm, table_hbm, out_hbm, idx_v, out_v, sem):
    wid = lax.axis_index("s") * 2 + lax.axis_index("c")
    pltpu.sync_copy(idx_hbm.at[wid], idx_v)

    pltpu.async_copy(table_hbm.at[idx_v], out_v, sem).wait()
    pltpu.sync_copy(out_v, out_hbm.at[wid])


def kernel(tokens_list, vocab_table):
    idx = tokens_list.reshape(_NW, _PER_W)
    out = _sc_gather(idx, vocab_table)
    return out.reshape(_B, _S)
